# Initial kernel scaffold; baseline (speedup 1.0000x reference)
import jax
import jax.numpy as jnp
from jax.experimental import pallas as pl


def kernel(x, table):
    raise NotImplementedError("write your pallas kernel here")


# speedup vs baseline: 2.1011x; 2.1011x over previous
import functools

import jax
import jax.numpy as jnp
from jax import lax
from jax.experimental import pallas as pl
from jax.experimental.pallas import tpu as pltpu
from jax.experimental.pallas import tpu_sc as plsc

VOCAB = 4000
EMBED = 64
BATCH = 16384
SEQ = 200

NC = 2
NS = 16
NW = NC * NS
BPW = BATCH // NW
NB = 8
NCHUNK = BPW // NB
EG = 16
NG = EMBED // EG
L_OFFS = tuple(range(0, SEQ - 16, 16)) + (SEQ - 16,)


def _sc_body(x_hbm, tbl_hbm, out_hbm, tblg, xbuf, obuf):
    wid = lax.axis_index("s") * NC + lax.axis_index("c")
    base_b = wid * BPW

    for g in range(NG):
        pltpu.sync_copy(tbl_hbm.at[g], tblg)

        def chunk_body(c, _, g=g):
            b0 = base_b + c * NB
            pltpu.sync_copy(x_hbm.at[pl.ds(b0, NB), :], xbuf)

            def b_body(bi, _):
                xv = [xbuf[bi, pl.ds(off, 16)] for off in L_OFFS]

                def e_body(e, _):
                    ev = jnp.full((16,), e, dtype=jnp.int32)
                    for j, off in enumerate(L_OFFS):
                        v = plsc.load_gather(tblg, [xv[j], ev])
                        obuf[bi, e, pl.ds(off, 16)] = v
                    return 0

                lax.fori_loop(0, EG, e_body, 0)
                return 0

            lax.fori_loop(0, NB, b_body, 0)
            pltpu.sync_copy(
                obuf, out_hbm.at[pl.ds(b0, NB), pl.ds(g * EG, EG), :]
            )
            return 0

        lax.fori_loop(0, NCHUNK, chunk_body, 0)


@functools.partial(jax.jit, static_argnames=())
def kernel(x, table):
    tbl4 = jnp.transpose(table.reshape(VOCAB, NG, EG), (1, 0, 2))
    x = x.astype(jnp.int32)

    run = pl.kernel(
        _sc_body,
        out_type=jax.ShapeDtypeStruct((BATCH, EMBED, SEQ), jnp.float32),
        mesh=plsc.VectorSubcoreMesh(core_axis_name="c", subcore_axis_name="s"),
        compiler_params=pltpu.CompilerParams(
            needs_layout_passes=False, use_tc_tiling_on_sc=False
        ),
        scratch_types=[
            pltpu.VMEM((VOCAB, EG), jnp.float32),
            pltpu.VMEM((NB, SEQ), jnp.int32),
            pltpu.VMEM((NB, EG, SEQ), jnp.float32),
        ],
    )
    return run(x, tbl4)


# speedup vs baseline: 2.8661x; 1.3641x over previous
import functools

import jax
import jax.numpy as jnp
from jax import lax
from jax.experimental import pallas as pl
from jax.experimental.pallas import tpu as pltpu
from jax.experimental.pallas import tpu_sc as plsc

VOCAB = 4000
EMBED = 64
BATCH = 16384
SEQ = 200

NC = 2
NS = 16
NW = NC * NS
BPW = BATCH // NW
NB = 8
NCHUNK = BPW // NB
EG = 16
NG = EMBED // EG
L_OFFS = tuple(range(0, SEQ - 16, 16)) + (SEQ - 16,)


def _sc_body(x_hbm, tbl_hbm, out_hbm, tblg, xbuf, obuf, osems, xsems):
    wid = lax.axis_index("s") * NC + lax.axis_index("c")
    base_b = wid * BPW

    def out_slice(b0, g):
        return out_hbm.at[pl.ds(b0, NB), pl.ds(g * EG, EG), :]

    for g in range(NG):
        pltpu.sync_copy(tbl_hbm.at[g], tblg)
        pltpu.async_copy(
            x_hbm.at[pl.ds(base_b, NB), :], xbuf.at[0], xsems.at[0]
        )

        def chunk_body(c, _, g=g):
            p = c & 1
            b0 = base_b + c * NB

            pltpu.make_async_copy(
                x_hbm.at[pl.ds(b0, NB), :], xbuf.at[p], xsems.at[p]
            ).wait()

            @pl.when(c + 1 < NCHUNK)
            def _():
                pltpu.async_copy(
                    x_hbm.at[pl.ds(b0 + NB, NB), :],
                    xbuf.at[1 - p],
                    xsems.at[1 - p],
                )

            @pl.when(c >= 2)
            def _():
                pltpu.make_async_copy(
                    obuf.at[p], out_slice(b0, g), osems.at[p]
                ).wait()

            def b_body(bi, _):
                xv16 = [xbuf[p, bi, pl.ds(off, 16)] << 4 for off in L_OFFS]
                for e in range(EG):
                    vals = [
                        plsc.load_gather(tblg, [xv16[j] + e])
                        for j in range(len(L_OFFS))
                    ]
                    for j, off in enumerate(L_OFFS):
                        obuf[p, bi, e, pl.ds(off, 16)] = vals[j]
                return 0

            lax.fori_loop(0, NB, b_body, 0)

            pltpu.async_copy(obuf.at[p], out_slice(b0, g), osems.at[p])
            return 0

        lax.fori_loop(0, NCHUNK, chunk_body, 0)

        for pp in range(2):
            b0d = base_b + (NCHUNK - 2 + pp) * NB
            pltpu.make_async_copy(
                obuf.at[pp], out_slice(b0d, g), osems.at[pp]
            ).wait()


@functools.partial(jax.jit, static_argnames=())
def kernel(x, table):
    tbl4 = jnp.transpose(table.reshape(VOCAB, NG, EG), (1, 0, 2)).reshape(
        NG, VOCAB * EG
    )
    x = x.astype(jnp.int32)

    run = pl.kernel(
        _sc_body,
        out_type=jax.ShapeDtypeStruct((BATCH, EMBED, SEQ), jnp.float32),
        mesh=plsc.VectorSubcoreMesh(core_axis_name="c", subcore_axis_name="s"),
        compiler_params=pltpu.CompilerParams(
            needs_layout_passes=False, use_tc_tiling_on_sc=False
        ),
        scratch_types=[
            pltpu.VMEM((VOCAB * EG,), jnp.float32),
            pltpu.VMEM((2, NB, SEQ), jnp.int32),
            pltpu.VMEM((2, NB, EG, SEQ), jnp.float32),
            pltpu.SemaphoreType.DMA((2,)),
            pltpu.SemaphoreType.DMA((2,)),
        ],
    )
    return run(x, tbl4)


# speedup vs baseline: 4.5799x; 1.5980x over previous
import functools

import jax
import jax.numpy as jnp
from jax import lax
from jax.experimental import pallas as pl
from jax.experimental.pallas import tpu as pltpu
from jax.experimental.pallas import tpu_sc as plsc

VOCAB = 4000
EMBED = 64
BATCH = 16384
SEQ = 200

NC = 2
NS = 16
NW = NC * NS
BPW = BATCH // NW
NB = 8
NCHUNK = BPW // NB
EG = 16
NG = EMBED // EG
EGP = EG + 1
L_OFFS = tuple(range(0, SEQ - 16, 16)) + (SEQ - 16,)


def _sc_body(x_hbm, tbl_hbm, out_hbm, tblg, xbuf, obuf, osems, xsems):
    wid = lax.axis_index("s") * NC + lax.axis_index("c")
    base_b = wid * BPW

    def out_slice(b0, g):
        return out_hbm.at[pl.ds(b0, NB), pl.ds(g * EG, EG), :]

    for g in range(NG):
        pltpu.sync_copy(tbl_hbm.at[g], tblg)
        pltpu.async_copy(
            x_hbm.at[pl.ds(base_b, NB), :], xbuf.at[0], xsems.at[0]
        )

        def chunk_body(c, _, g=g):
            p = c & 1
            b0 = base_b + c * NB

            pltpu.make_async_copy(
                x_hbm.at[pl.ds(b0, NB), :], xbuf.at[p], xsems.at[p]
            ).wait()

            @pl.when(c + 1 < NCHUNK)
            def _():
                pltpu.async_copy(
                    x_hbm.at[pl.ds(b0 + NB, NB), :],
                    xbuf.at[1 - p],
                    xsems.at[1 - p],
                )

            @pl.when(c >= 2)
            def _():
                pltpu.make_async_copy(
                    obuf.at[p], out_slice(b0, g), osems.at[p]
                ).wait()

            def b_body(bi, _):
                xv16 = [
                    (lambda v: (v << 4) + v)(xbuf[p, bi, pl.ds(off, 16)])
                    for off in L_OFFS
                ]
                for e in range(EG):
                    vals = [
                        plsc.load_gather(tblg, [xv16[j] + e])
                        for j in range(len(L_OFFS))
                    ]
                    for j, off in enumerate(L_OFFS):
                        obuf[p, bi, e, pl.ds(off, 16)] = vals[j]
                return 0

            lax.fori_loop(0, NB, b_body, 0)

            pltpu.async_copy(obuf.at[p], out_slice(b0, g), osems.at[p])
            return 0

        lax.fori_loop(0, NCHUNK, chunk_body, 0)

        for pp in range(2):
            b0d = base_b + (NCHUNK - 2 + pp) * NB
            pltpu.make_async_copy(
                obuf.at[pp], out_slice(b0d, g), osems.at[pp]
            ).wait()


@functools.partial(jax.jit, static_argnames=())
def kernel(x, table):
    tbl4 = jnp.pad(
        jnp.transpose(table.reshape(VOCAB, NG, EG), (1, 0, 2)),
        ((0, 0), (0, 0), (0, 1)),
    ).reshape(NG, VOCAB * EGP)
    x = x.astype(jnp.int32)

    run = pl.kernel(
        _sc_body,
        out_type=jax.ShapeDtypeStruct((BATCH, EMBED, SEQ), jnp.float32),
        mesh=plsc.VectorSubcoreMesh(core_axis_name="c", subcore_axis_name="s"),
        compiler_params=pltpu.CompilerParams(
            needs_layout_passes=False, use_tc_tiling_on_sc=False
        ),
        scratch_types=[
            pltpu.VMEM((VOCAB * EGP,), jnp.float32),
            pltpu.VMEM((2, NB, SEQ), jnp.int32),
            pltpu.VMEM((2, NB, EG, SEQ), jnp.float32),
            pltpu.SemaphoreType.DMA((2,)),
            pltpu.SemaphoreType.DMA((2,)),
        ],
    )
    return run(x, tbl4)


# speedup vs baseline: 21.5048x; 4.6954x over previous
import functools

import jax
import jax.numpy as jnp
from jax import lax
from jax.experimental import pallas as pl
from jax.experimental.pallas import tpu as pltpu
from jax.experimental.pallas import tpu_sc as plsc

VOCAB = 4000
EMBED = 64
BATCH = 16384
SEQ = 200

NC = 2
NS = 16
NW = NC * NS
EG = 16
NG = EMBED // EG
EGP = EG + 1
LT = SEQ // 8
BT = BATCH // 128
BT_PER_W = BT // NW
NCHUNK = LT


def _sc_body(x_hbm, tbl_hbm, out_hbm, tblg, xbuf, obuf, osems, xsems):
    wid = lax.axis_index("s") * NC + lax.axis_index("c")

    def g_body(g, _):
        pltpu.sync_copy(tbl_hbm.at[g], tblg)

        def i_body(i, _):
            bt = wid * BT_PER_W + i

            pltpu.async_copy(x_hbm.at[0, bt], xbuf.at[0], xsems.at[0])

            def chunk_body(lt, _):
                p = lt & 1

                pltpu.make_async_copy(
                    x_hbm.at[lt, bt], xbuf.at[p], xsems.at[p]
                ).wait()

                @pl.when(lt + 1 < NCHUNK)
                def _():
                    pltpu.async_copy(
                        x_hbm.at[lt + 1, bt], xbuf.at[1 - p], xsems.at[1 - p]
                    )

                @pl.when(lt >= 2)
                def _():
                    pltpu.make_async_copy(
                        obuf.at[p],
                        out_hbm.at[pl.ds(g * EG, EG), lt, bt],
                        osems.at[p],
                    ).wait()

                def ls_body(ls, _):
                    xv17 = [
                        (lambda v: (v << 4) + v)(
                            xbuf[p, ls, pl.ds(bv * 16, 16)]
                        )
                        for bv in range(8)
                    ]
                    for e in range(EG):
                        vals = [
                            plsc.load_gather(tblg, [xv17[bv] + e])
                            for bv in range(8)
                        ]
                        for bv in range(8):
                            obuf[p, e, ls, pl.ds(bv * 16, 16)] = vals[bv]
                    return 0

                lax.fori_loop(0, 8, ls_body, 0)

                pltpu.async_copy(
                    obuf.at[p],
                    out_hbm.at[pl.ds(g * EG, EG), lt, bt],
                    osems.at[p],
                )
                return 0

            lax.fori_loop(0, NCHUNK, chunk_body, 0)

            for pp in range(2):
                pltpu.make_async_copy(
                    obuf.at[pp],
                    out_hbm.at[pl.ds(g * EG, EG), 0, bt],
                    osems.at[pp],
                ).wait()
            return 0

        lax.fori_loop(0, BT_PER_W, i_body, 0)
        return 0

    lax.fori_loop(0, NG, g_body, 0)


@functools.partial(jax.jit, static_argnames=())
def kernel(x, table):
    tbl4 = jnp.pad(
        jnp.transpose(table.reshape(VOCAB, NG, EG), (1, 0, 2)),
        ((0, 0), (0, 0), (0, 1)),
    ).reshape(NG, VOCAB * EGP)
    xt = jnp.transpose(
        x.astype(jnp.int32).T.reshape(LT, 8, BT, 128), (0, 2, 1, 3)
    )

    run = pl.kernel(
        _sc_body,
        out_type=jax.ShapeDtypeStruct((EMBED, LT, BT, 8, 128), jnp.float32),
        mesh=plsc.VectorSubcoreMesh(core_axis_name="c", subcore_axis_name="s"),
        compiler_params=pltpu.CompilerParams(
            needs_layout_passes=False, use_tc_tiling_on_sc=False
        ),
        scratch_types=[
            pltpu.VMEM((VOCAB * EGP,), jnp.float32),
            pltpu.VMEM((2, 8, 128), jnp.int32),
            pltpu.VMEM((2, EG, 8, 128), jnp.float32),
            pltpu.SemaphoreType.DMA((2,)),
            pltpu.SemaphoreType.DMA((2,)),
        ],
    )
    out5 = run(xt, tbl4)
    return jnp.transpose(out5, (2, 4, 0, 1, 3)).reshape(BATCH, EMBED, SEQ)


# speedup vs baseline: 27.7913x; 1.2923x over previous
import functools

import jax
import jax.numpy as jnp
from jax import lax
from jax.experimental import pallas as pl
from jax.experimental.pallas import tpu as pltpu
from jax.experimental.pallas import tpu_sc as plsc

VOCAB = 4000
EMBED = 64
BATCH = 16384
SEQ = 200

NC = 2
NS = 16
NW = NC * NS
EG = 16
NG = EMBED // EG
EGP = EG + 1
LT = SEQ // 8
BT = BATCH // 128
BT_PER_W = BT // NW
NCHUNK = LT


def _sc_body(x_hbm, tbl_hbm, out_hbm, tblg, xbuf, obuf, osems, xsems):
    wid = lax.axis_index("s") * NC + lax.axis_index("c")

    def g_body(g, _):
        pltpu.sync_copy(tbl_hbm.at[g], tblg)

        def i_body(i, _):
            bt = wid * BT_PER_W + i

            pltpu.async_copy(x_hbm.at[0, bt], xbuf.at[0], xsems.at[0])

            def chunk_body(lt, _):
                p = lt & 1

                pltpu.make_async_copy(
                    x_hbm.at[lt, bt], xbuf.at[p], xsems.at[p]
                ).wait()

                @pl.when(lt + 1 < NCHUNK)
                def _():
                    pltpu.async_copy(
                        x_hbm.at[lt + 1, bt], xbuf.at[1 - p], xsems.at[1 - p]
                    )

                @pl.when(lt >= 2)
                def _():
                    pltpu.make_async_copy(
                        obuf.at[p],
                        out_hbm.at[pl.ds(g * EG, EG), lt, bt],
                        osems.at[p],
                    ).wait()

                def ls_body(ls, _):
                    xv17 = [
                        (lambda v: (v << 4) + v)(
                            xbuf[p, ls, pl.ds(bv * 16, 16)]
                        )
                        for bv in range(8)
                    ]
                    vals = [
                        plsc.load_gather(tblg, [xv17[bv]]) for bv in range(8)
                    ]
                    for e in range(1, EG):
                        nvals = []
                        for bv in range(8):
                            nvals.append(
                                plsc.load_gather(tblg, [xv17[bv] + e])
                            )
                            obuf[p, e - 1, ls, pl.ds(bv * 16, 16)] = vals[bv]
                        vals = nvals
                    for bv in range(8):
                        obuf[p, EG - 1, ls, pl.ds(bv * 16, 16)] = vals[bv]
                    return 0

                lax.fori_loop(0, 8, ls_body, 0)

                pltpu.async_copy(
                    obuf.at[p],
                    out_hbm.at[pl.ds(g * EG, EG), lt, bt],
                    osems.at[p],
                )
                return 0

            lax.fori_loop(0, NCHUNK, chunk_body, 0)

            for pp in range(2):
                pltpu.make_async_copy(
                    obuf.at[pp],
                    out_hbm.at[pl.ds(g * EG, EG), 0, bt],
                    osems.at[pp],
                ).wait()
            return 0

        lax.fori_loop(0, BT_PER_W, i_body, 0)
        return 0

    lax.fori_loop(0, NG, g_body, 0)


@functools.partial(jax.jit, static_argnames=())
def kernel(x, table):
    tbl4 = jnp.pad(
        jnp.transpose(table.reshape(VOCAB, NG, EG), (1, 0, 2)),
        ((0, 0), (0, 0), (0, 1)),
    ).reshape(NG, VOCAB * EGP)
    xt = jnp.transpose(
        x.astype(jnp.int32).T.reshape(LT, 8, BT, 128), (0, 2, 1, 3)
    )

    run = pl.kernel(
        _sc_body,
        out_type=jax.ShapeDtypeStruct((EMBED, LT, BT, 8, 128), jnp.float32),
        mesh=plsc.VectorSubcoreMesh(core_axis_name="c", subcore_axis_name="s"),
        compiler_params=pltpu.CompilerParams(
            needs_layout_passes=False, use_tc_tiling_on_sc=False
        ),
        scratch_types=[
            pltpu.VMEM((VOCAB * EGP,), jnp.float32),
            pltpu.VMEM((2, 8, 128), jnp.int32),
            pltpu.VMEM((2, EG, 8, 128), jnp.float32),
            pltpu.SemaphoreType.DMA((2,)),
            pltpu.SemaphoreType.DMA((2,)),
        ],
    )
    out5 = run(xt, tbl4)
    return jnp.transpose(out5, (2, 4, 0, 1, 3)).reshape(BATCH, EMBED, SEQ)


# speedup vs baseline: 27.8318x; 1.0015x over previous
import functools

import jax
import jax.numpy as jnp
from jax import lax
from jax.experimental import pallas as pl
from jax.experimental.pallas import tpu as pltpu
from jax.experimental.pallas import tpu_sc as plsc

VOCAB = 4000
EMBED = 64
BATCH = 16384
SEQ = 200

NC = 2
NS = 16
NW = NC * NS
EG = 16
NG = EMBED // EG
EGP = EG + 1
LT = SEQ // 8
BT = BATCH // 128
BT_PER_W = BT // NW
NCHUNK = LT


def _sc_body(x_hbm, tbl_hbm, out_hbm, tblg, xbuf, obuf, osems, xsems):
    wid = lax.axis_index("s") * NC + lax.axis_index("c")

    def g_body(g, _):
        pltpu.sync_copy(tbl_hbm.at[g], tblg)

        def i_body(i, _):
            bt = wid * BT_PER_W + i

            pltpu.async_copy(x_hbm.at[0, bt], xbuf.at[0], xsems.at[0])

            def chunk_body(lt, _):
                p = lt & 1
                po = lax.rem(lt, 3)

                pltpu.make_async_copy(
                    x_hbm.at[lt, bt], xbuf.at[p], xsems.at[p]
                ).wait()

                @pl.when(lt + 1 < NCHUNK)
                def _():
                    pltpu.async_copy(
                        x_hbm.at[lt + 1, bt], xbuf.at[1 - p], xsems.at[1 - p]
                    )

                @pl.when(lt >= 3)
                def _():
                    pltpu.make_async_copy(
                        obuf.at[po],
                        out_hbm.at[pl.ds(g * EG, EG), lt, bt],
                        osems.at[po],
                    ).wait()

                def ls_body(ls):
                    xv17 = [
                        (lambda v: (v << 4) + v)(
                            xbuf[p, ls, pl.ds(bv * 16, 16)]
                        )
                        for bv in range(8)
                    ]
                    vals = [
                        plsc.load_gather(tblg, [xv17[bv]]) for bv in range(8)
                    ]
                    for e in range(1, EG):
                        nvals = []
                        for bv in range(8):
                            nvals.append(
                                plsc.load_gather(tblg, [xv17[bv] + e])
                            )
                            obuf[po, e - 1, ls, pl.ds(bv * 16, 16)] = vals[bv]
                        vals = nvals
                    for bv in range(8):
                        obuf[po, EG - 1, ls, pl.ds(bv * 16, 16)] = vals[bv]

                plsc.parallel_loop(0, 8, unroll=2)(ls_body)

                pltpu.async_copy(
                    obuf.at[po],
                    out_hbm.at[pl.ds(g * EG, EG), lt, bt],
                    osems.at[po],
                )
                return 0

            lax.fori_loop(0, NCHUNK, chunk_body, 0)

            for pp in range(3):
                pltpu.make_async_copy(
                    obuf.at[pp],
                    out_hbm.at[pl.ds(g * EG, EG), 0, bt],
                    osems.at[pp],
                ).wait()
            return 0

        lax.fori_loop(0, BT_PER_W, i_body, 0)
        return 0

    lax.fori_loop(0, NG, g_body, 0)


@functools.partial(jax.jit, static_argnames=())
def kernel(x, table):
    tbl4 = jnp.pad(
        jnp.transpose(table.reshape(VOCAB, NG, EG), (1, 0, 2)),
        ((0, 0), (0, 0), (0, 1)),
    ).reshape(NG, VOCAB * EGP)
    xt = jnp.transpose(
        x.astype(jnp.int32).T.reshape(LT, 8, BT, 128), (0, 2, 1, 3)
    )

    run = pl.kernel(
        _sc_body,
        out_type=jax.ShapeDtypeStruct((EMBED, LT, BT, 8, 128), jnp.float32),
        mesh=plsc.VectorSubcoreMesh(core_axis_name="c", subcore_axis_name="s"),
        compiler_params=pltpu.CompilerParams(
            needs_layout_passes=False, use_tc_tiling_on_sc=False
        ),
        scratch_types=[
            pltpu.VMEM((VOCAB * EGP,), jnp.float32),
            pltpu.VMEM((2, 8, 128), jnp.int32),
            pltpu.VMEM((3, EG, 8, 128), jnp.float32),
            pltpu.SemaphoreType.DMA((3,)),
            pltpu.SemaphoreType.DMA((2,)),
        ],
    )
    out5 = run(xt, tbl4)
    return jnp.transpose(out5, (2, 4, 0, 1, 3)).reshape(BATCH, EMBED, SEQ)
